# baseline (speedup 1.0000x reference)
import jax
import jax.numpy as jnp
from jax import lax
from jax.experimental import pallas as pl
from jax.experimental.pallas import tpu as pltpu

_B, _T, _C = 16, 1024, 999
_N = _B * _T
_OPS = 8
_ROWS = 512
_STEPS = _N // (_OPS * _ROWS)


def _body(*refs):
    x_refs = refs[:_OPS]
    t_refs = refs[_OPS:2 * _OPS]
    o_ref = refs[2 * _OPS]
    col = lax.broadcasted_iota(jnp.int32, (_ROWS, _C), 1)
    part = jnp.float32(0.0)
    for x_ref, t_ref in zip(x_refs, t_refs):
        x = x_ref[...]
        t = t_ref[...]
        hit = col == (t - 1)
        part += jnp.sum(x * x) - 2.0 * jnp.sum(jnp.where(hit, x, 0.0))

    @pl.when(pl.program_id(0) == 0)
    def _():
        o_ref[0, 0] = 0.0

    o_ref[0, 0] += part


def kernel(rel_ress, targets, mask):
    del mask
    x = rel_ress.reshape(_N, _C)
    t_col = targets.astype(jnp.int32).reshape(_N, 1)
    x_specs = [
        pl.BlockSpec((_ROWS, _C), lambda i, k=k: (i + k * _STEPS, 0))
        for k in range(_OPS)
    ]
    t_specs = [
        pl.BlockSpec((_ROWS, 1), lambda i, k=k: (i + k * _STEPS, 0))
        for k in range(_OPS)
    ]
    out = pl.pallas_call(
        _body,
        grid=(_STEPS,),
        in_specs=x_specs + t_specs,
        out_specs=pl.BlockSpec(memory_space=pltpu.SMEM),
        out_shape=jax.ShapeDtypeStruct((1, 1), jnp.float32),
    )(*([x] * _OPS + [t_col] * _OPS))
    return (out[0, 0] + jnp.float32(_N)) / jnp.float32(_N * (_C + 1))
